# Initial kernel scaffold; baseline (speedup 1.0000x reference)
import jax
import jax.numpy as jnp
from jax.experimental import pallas as pl


def kernel(x):
    raise NotImplementedError("write your pallas kernel here")


# speedup vs baseline: 19.3762x; 19.3762x over previous
import functools

import jax
import jax.numpy as jnp
import numpy as np
from jax import lax
from jax.experimental import pallas as pl
from jax.experimental.pallas import tpu as pltpu
from jax.experimental.pallas import tpu_sc as plsc

_NX0, _NX1 = 600, 300
_NCELL = _NX0 * _NX1
_NCAM, _NCH = 6, 64
_FH, _FW = 32, 64
_PIX = _FH * _FW
_NSRC = _NCAM * _PIX
_D = 71

_CHUNK = 4000
_NCHUNK = _NCELL // _CHUNK
_GROUPS = _CHUNK // 16
_TBL = _NSRC + 16


def _winner_map() -> np.ndarray:
    ds = np.arange(4.0, 75.0, 1.0, dtype=np.float32).reshape(-1, 1, 1) * np.ones(
        (1, _FH, _FW), np.float32)
    xs = np.linspace(0, 1023, _FW, dtype=np.float32).reshape(1, 1, _FW) * np.ones(
        (_D, _FH, 1), np.float32)
    ys = np.linspace(0, 511, _FH, dtype=np.float32).reshape(1, _FH, 1) * np.ones(
        (_D, 1, _FW), np.float32)

    matrix = np.asarray([[2019.613635, 1745.881668, -111.4337968, -419.9388818],
                         [26.01936737, 870.7969811, -2038.300785, -120.9971104],
                         [0.02443084799, 0.997614078, -0.06457000164, -0.006415358346]])
    m = np.vstack([matrix, np.asarray([0.0, 0.0, 0.0, 1.0])])
    inv_m = np.linalg.inv(m).astype(np.float32)

    c1700 = np.float32(1700.0)
    c512 = np.float32(512.0)
    c3517 = np.float32(3517.0)
    c1024 = np.float32(1024.0)
    p4 = np.stack([
        (xs * ds) * c1700 / c512,
        (ys * ds) * c3517 / c1024,
        ds,
        np.ones_like(ds),
    ], -1).reshape(-1, 4)

    geom = np.asarray(jnp.dot(jnp.asarray(p4), jnp.asarray(inv_m.T)))[:, :3]

    dx = np.array([0.25, 0.25, 20.0], dtype=np.float32)
    bx = np.array([-75.0 + 0.125, 0.0 + 0.125, -10.0 + 10.0], dtype=np.float32)
    gf = ((geom - (bx - dx / np.float32(2.0))) / dx).astype(np.int32)

    kept = ((gf[:, 0] >= 0) & (gf[:, 0] < _NX0)
            & (gf[:, 1] >= 0) & (gf[:, 1] < _NX1)
            & (gf[:, 2] >= 0) & (gf[:, 2] < 1))
    rank = gf[:, 0].astype(np.int64) * _NX1 + gf[:, 1]

    q = np.arange(_D * _PIX, dtype=np.int64)
    winner_q = np.full(_NCELL, -1, dtype=np.int64)
    np.maximum.at(winner_q, rank[kept], q[kept])

    d_win = winner_q // _PIX
    cam = (5 * _D + d_win) % _NCAM
    src = (cam * _PIX + winner_q % _PIX).astype(np.int32)
    idx_map = np.where(winner_q >= 0, src, np.int32(_NSRC)).astype(np.int32)
    return idx_map


_IDX_MAP = _winner_map()

@functools.cache
def _bev_gather_kernel():
    mesh = plsc.VectorSubcoreMesh(
        core_axis_name="c", subcore_axis_name="s", num_cores=2, num_subcores=16)
    return functools.partial(
        pl.kernel,
        out_type=jax.ShapeDtypeStruct((_NCH * _NCELL,), jnp.float32),
        mesh=mesh,
        scratch_types=[
            pltpu.VMEM((_TBL,), jnp.float32),
            pltpu.VMEM((_TBL,), jnp.float32),
            pltpu.VMEM((_CHUNK,), jnp.int32),
            pltpu.VMEM((_CHUNK,), jnp.float32),
            pltpu.VMEM((_CHUNK,), jnp.float32),
        ],
        compiler_params=pltpu.CompilerParams(needs_layout_passes=False),
    )(_bev_gather_body)


def _bev_gather_body(x_hbm, idx_hbm, out_hbm, tbl_a, tbl_b, idx_v, out_a, out_b):
    wid = lax.axis_index("s") * 2 + lax.axis_index("c")
    c0 = wid * 2

    for cam in range(_NCAM):
        off_a = pl.multiple_of((cam * _NCH + c0) * _PIX, _PIX)
        off_b = pl.multiple_of((cam * _NCH + c0 + 1) * _PIX, _PIX)
        pltpu.sync_copy(x_hbm.at[pl.ds(off_a, _PIX)],
                        tbl_a.at[pl.ds(cam * _PIX, _PIX)])
        pltpu.sync_copy(x_hbm.at[pl.ds(off_b, _PIX)],
                        tbl_b.at[pl.ds(cam * _PIX, _PIX)])
    zeros = jnp.zeros((16,), jnp.float32)
    tbl_a[pl.ds(_NSRC, 16)] = zeros
    tbl_b[pl.ds(_NSRC, 16)] = zeros

    out_base_a = c0 * _NCELL
    out_base_b = (c0 + 1) * _NCELL

    def chunk_body(k, carry):
        off = pl.multiple_of(k * _CHUNK, _CHUNK)
        pltpu.sync_copy(idx_hbm.at[pl.ds(off, _CHUNK)], idx_v)

        def group(g, carry2):
            iv = idx_v[pl.ds(g * 16, 16)]
            out_a[pl.ds(g * 16, 16)] = plsc.load_gather(tbl_a, [iv])
            out_b[pl.ds(g * 16, 16)] = plsc.load_gather(tbl_b, [iv])
            return carry2

        lax.fori_loop(0, _GROUPS, group, 0)
        pltpu.sync_copy(out_a, out_hbm.at[pl.ds(pl.multiple_of(out_base_a + off, 8), _CHUNK)])
        pltpu.sync_copy(out_b, out_hbm.at[pl.ds(pl.multiple_of(out_base_b + off, 8), _CHUNK)])
        return carry

    lax.fori_loop(0, _NCHUNK, chunk_body, 0)


def kernel(x):
    x1 = jnp.reshape(jnp.asarray(x, jnp.float32), (_NCAM * _NCH * _PIX,))
    out = _bev_gather_kernel()(x1, jnp.asarray(_IDX_MAP))
    return jnp.reshape(out, (1, _NCH, _NX0, _NX1))


# speedup vs baseline: 20.6126x; 1.0638x over previous
import functools

import jax
import jax.numpy as jnp
import numpy as np
from jax import lax
from jax.experimental import pallas as pl
from jax.experimental.pallas import tpu as pltpu
from jax.experimental.pallas import tpu_sc as plsc

_NX0, _NX1 = 600, 300
_NCELL = _NX0 * _NX1
_NCAM, _NCH = 6, 64
_FH, _FW = 32, 64
_PIX = _FH * _FW
_NSRC = _NCAM * _PIX
_D = 71

_CHUNK = 12000
_NCHUNK = _NCELL // _CHUNK
_UNROLL = 10
_GROUPS = _CHUNK // (16 * _UNROLL)
_TBL = _NSRC + 16


def _winner_map() -> np.ndarray:
    ds = np.arange(4.0, 75.0, 1.0, dtype=np.float32).reshape(-1, 1, 1) * np.ones(
        (1, _FH, _FW), np.float32)
    xs = np.linspace(0, 1023, _FW, dtype=np.float32).reshape(1, 1, _FW) * np.ones(
        (_D, _FH, 1), np.float32)
    ys = np.linspace(0, 511, _FH, dtype=np.float32).reshape(1, _FH, 1) * np.ones(
        (_D, 1, _FW), np.float32)

    matrix = np.asarray([[2019.613635, 1745.881668, -111.4337968, -419.9388818],
                         [26.01936737, 870.7969811, -2038.300785, -120.9971104],
                         [0.02443084799, 0.997614078, -0.06457000164, -0.006415358346]])
    m = np.vstack([matrix, np.asarray([0.0, 0.0, 0.0, 1.0])])
    inv_m = np.linalg.inv(m).astype(np.float32)

    c1700 = np.float32(1700.0)
    c512 = np.float32(512.0)
    c3517 = np.float32(3517.0)
    c1024 = np.float32(1024.0)
    p4 = np.stack([
        (xs * ds) * c1700 / c512,
        (ys * ds) * c3517 / c1024,
        ds,
        np.ones_like(ds),
    ], -1).reshape(-1, 4)

    geom = np.asarray(jnp.dot(jnp.asarray(p4), jnp.asarray(inv_m.T)))[:, :3]

    dx = np.array([0.25, 0.25, 20.0], dtype=np.float32)
    bx = np.array([-75.0 + 0.125, 0.0 + 0.125, -10.0 + 10.0], dtype=np.float32)
    gf = ((geom - (bx - dx / np.float32(2.0))) / dx).astype(np.int32)

    kept = ((gf[:, 0] >= 0) & (gf[:, 0] < _NX0)
            & (gf[:, 1] >= 0) & (gf[:, 1] < _NX1)
            & (gf[:, 2] >= 0) & (gf[:, 2] < 1))
    rank = gf[:, 0].astype(np.int64) * _NX1 + gf[:, 1]

    q = np.arange(_D * _PIX, dtype=np.int64)
    winner_q = np.full(_NCELL, -1, dtype=np.int64)
    np.maximum.at(winner_q, rank[kept], q[kept])

    d_win = winner_q // _PIX
    cam = (5 * _D + d_win) % _NCAM
    src = (cam * _PIX + winner_q % _PIX).astype(np.int32)
    idx_map = np.where(winner_q >= 0, src, np.int32(_NSRC)).astype(np.int32)
    return idx_map


_IDX_MAP = _winner_map()

@functools.cache
def _bev_gather_kernel():
    mesh = plsc.VectorSubcoreMesh(
        core_axis_name="c", subcore_axis_name="s", num_cores=2, num_subcores=16)
    return functools.partial(
        pl.kernel,
        out_type=jax.ShapeDtypeStruct((_NCH * _NCELL,), jnp.float32),
        mesh=mesh,
        scratch_types=[
            pltpu.VMEM((_TBL,), jnp.float32),
            pltpu.VMEM((_TBL,), jnp.float32),
            pltpu.VMEM((_CHUNK,), jnp.int32),
            pltpu.VMEM((_CHUNK,), jnp.int32),
            pltpu.VMEM((_CHUNK,), jnp.float32),
            pltpu.VMEM((_CHUNK,), jnp.float32),
            pltpu.VMEM((_CHUNK,), jnp.float32),
            pltpu.VMEM((_CHUNK,), jnp.float32),
            pltpu.SemaphoreType.DMA,
            pltpu.SemaphoreType.DMA,
            pltpu.SemaphoreType.DMA,
            pltpu.SemaphoreType.DMA,
            pltpu.SemaphoreType.DMA,
        ],
        compiler_params=pltpu.CompilerParams(needs_layout_passes=False),
    )(_bev_gather_body)


def _bev_gather_body(x_hbm, idx_hbm, out_hbm, tbl_a, tbl_b,
                     idx0, idx1, outa0, outa1, outb0, outb1,
                     sem_in0, sem_in1, sem_out0, sem_out1, sem_tbl):
    idx_bufs = (idx0, idx1)
    outa_bufs = (outa0, outa1)
    outb_bufs = (outb0, outb1)
    sem_ins = (sem_in0, sem_in1)
    sem_outs = (sem_out0, sem_out1)
    wid = lax.axis_index("s") * 2 + lax.axis_index("c")
    c0 = wid * 2

    staging = []
    for cam in range(_NCAM):
        off_a = pl.multiple_of((cam * _NCH + c0) * _PIX, _PIX)
        off_b = pl.multiple_of((cam * _NCH + c0 + 1) * _PIX, _PIX)
        for off, tbl in ((off_a, tbl_a), (off_b, tbl_b)):
            cp = pltpu.make_async_copy(
                x_hbm.at[pl.ds(off, _PIX)],
                tbl.at[pl.ds(cam * _PIX, _PIX)], sem_tbl)
            cp.start()
            staging.append(cp)
    zeros = jnp.zeros((16,), jnp.float32)
    tbl_a[pl.ds(_NSRC, 16)] = zeros
    tbl_b[pl.ds(_NSRC, 16)] = zeros

    base_a = pl.multiple_of(c0 * _NCELL, 8)
    base_b = pl.multiple_of((c0 + 1) * _NCELL, 8)

    def in_copy(k, b):
        return pltpu.make_async_copy(
            idx_hbm.at[pl.ds(k * _CHUNK, _CHUNK)], idx_bufs[b], sem_ins[b])

    def out_copy(ref, b, base, k):
        return pltpu.make_async_copy(
            ref, out_hbm.at[pl.ds(base + k * _CHUNK, _CHUNK)], sem_outs[b])

    in_copy(0, 0).start()
    for cp in staging:
        cp.wait()

    for k in range(_NCHUNK):
        b = k & 1
        if k + 1 < _NCHUNK:
            in_copy(k + 1, 1 - b).start()
        in_copy(k, b).wait()
        if k >= 2:
            out_copy(outa_bufs[b], b, base_a, k - 2).wait()
            out_copy(outb_bufs[b], b, base_b, k - 2).wait()
        idx_v, out_a, out_b = idx_bufs[b], outa_bufs[b], outb_bufs[b]

        def step(g, carry):
            gbase = g * (16 * _UNROLL)
            for u in range(_UNROLL):
                o = gbase + u * 16
                iv = idx_v[pl.ds(o, 16)]
                out_a[pl.ds(o, 16)] = plsc.load_gather(tbl_a, [iv])
                out_b[pl.ds(o, 16)] = plsc.load_gather(tbl_b, [iv])
            return carry

        lax.fori_loop(0, _GROUPS, step, 0)
        out_copy(out_a, b, base_a, k).start()
        out_copy(out_b, b, base_b, k).start()

    for k in (_NCHUNK - 2, _NCHUNK - 1):
        b = k & 1
        out_copy(outa_bufs[b], b, base_a, k).wait()
        out_copy(outb_bufs[b], b, base_b, k).wait()


def kernel(x):
    x1 = jnp.reshape(jnp.asarray(x, jnp.float32), (_NCAM * _NCH * _PIX,))
    out = _bev_gather_kernel()(x1, jnp.asarray(_IDX_MAP))
    return jnp.reshape(out, (1, _NCH, _NX0, _NX1))


# speedup vs baseline: 26.1370x; 1.2680x over previous
import functools

import jax
import jax.numpy as jnp
import numpy as np
from jax import lax
from jax.experimental import pallas as pl
from jax.experimental.pallas import tpu as pltpu
from jax.experimental.pallas import tpu_sc as plsc

_NX0, _NX1 = 600, 300
_NCELL = _NX0 * _NX1
_NCAM, _NCH = 6, 64
_FH, _FW = 32, 64
_PIX = _FH * _FW
_NSRC = _NCAM * _PIX
_D = 71

_CHUNK = 12000
_NCHUNK = _NCELL // _CHUNK
_UNROLL = 10
_GROUPS = _CHUNK // (16 * _UNROLL)
_TBL = _NSRC + 16


def _winner_map() -> np.ndarray:
    ds = np.arange(4.0, 75.0, 1.0, dtype=np.float32).reshape(-1, 1, 1) * np.ones(
        (1, _FH, _FW), np.float32)
    xs = np.linspace(0, 1023, _FW, dtype=np.float32).reshape(1, 1, _FW) * np.ones(
        (_D, _FH, 1), np.float32)
    ys = np.linspace(0, 511, _FH, dtype=np.float32).reshape(1, _FH, 1) * np.ones(
        (_D, 1, _FW), np.float32)

    matrix = np.asarray([[2019.613635, 1745.881668, -111.4337968, -419.9388818],
                         [26.01936737, 870.7969811, -2038.300785, -120.9971104],
                         [0.02443084799, 0.997614078, -0.06457000164, -0.006415358346]])
    m = np.vstack([matrix, np.asarray([0.0, 0.0, 0.0, 1.0])])
    inv_m = np.linalg.inv(m).astype(np.float32)

    c1700 = np.float32(1700.0)
    c512 = np.float32(512.0)
    c3517 = np.float32(3517.0)
    c1024 = np.float32(1024.0)
    p4 = np.stack([
        (xs * ds) * c1700 / c512,
        (ys * ds) * c3517 / c1024,
        ds,
        np.ones_like(ds),
    ], -1).reshape(-1, 4)

    geom = np.asarray(jnp.dot(jnp.asarray(p4), jnp.asarray(inv_m.T)))[:, :3]

    dx = np.array([0.25, 0.25, 20.0], dtype=np.float32)
    bx = np.array([-75.0 + 0.125, 0.0 + 0.125, -10.0 + 10.0], dtype=np.float32)
    gf = ((geom - (bx - dx / np.float32(2.0))) / dx).astype(np.int32)

    kept = ((gf[:, 0] >= 0) & (gf[:, 0] < _NX0)
            & (gf[:, 1] >= 0) & (gf[:, 1] < _NX1)
            & (gf[:, 2] >= 0) & (gf[:, 2] < 1))
    rank = gf[:, 0].astype(np.int64) * _NX1 + gf[:, 1]

    q = np.arange(_D * _PIX, dtype=np.int64)
    winner_q = np.full(_NCELL, -1, dtype=np.int64)
    np.maximum.at(winner_q, rank[kept], q[kept])

    d_win = winner_q // _PIX
    cam = (5 * _D + d_win) % _NCAM
    src = (cam * _PIX + winner_q % _PIX).astype(np.int32)
    idx_map = np.where(winner_q >= 0, src, np.int32(_NSRC)).astype(np.int32)
    return idx_map


_IDX_MAP = _winner_map()

@functools.cache
def _bev_gather_kernel():
    mesh = plsc.VectorSubcoreMesh(
        core_axis_name="c", subcore_axis_name="s", num_cores=2, num_subcores=16)
    return functools.partial(
        pl.kernel,
        out_type=jax.ShapeDtypeStruct((_NCH * _NCELL,), jnp.float32),
        mesh=mesh,
        scratch_types=[
            pltpu.VMEM((_TBL,), jnp.float32),
            pltpu.VMEM((_TBL,), jnp.float32),
            pltpu.VMEM((_CHUNK,), jnp.int32),
            pltpu.VMEM((_CHUNK,), jnp.int32),
            pltpu.VMEM((_CHUNK,), jnp.float32),
            pltpu.VMEM((_CHUNK,), jnp.float32),
            pltpu.VMEM((_CHUNK,), jnp.float32),
            pltpu.VMEM((_CHUNK,), jnp.float32),
            pltpu.SemaphoreType.DMA,
            pltpu.SemaphoreType.DMA,
            pltpu.SemaphoreType.DMA,
            pltpu.SemaphoreType.DMA,
            pltpu.SemaphoreType.DMA,
        ],
        compiler_params=pltpu.CompilerParams(needs_layout_passes=False),
    )(_bev_gather_body)


def _bev_gather_body(x_hbm, idx_hbm, out_hbm, tbl_a, tbl_b,
                     idx0, idx1, outa0, outa1, outb0, outb1,
                     sem_in0, sem_in1, sem_out0, sem_out1, sem_tbl):
    idx_bufs = (idx0, idx1)
    outa_bufs = (outa0, outa1)
    outb_bufs = (outb0, outb1)
    sem_ins = (sem_in0, sem_in1)
    sem_outs = (sem_out0, sem_out1)
    wid = lax.axis_index("s") * 2 + lax.axis_index("c")
    c0 = wid * 2

    staging = []
    for cam in range(_NCAM):
        off_a = pl.multiple_of((cam * _NCH + c0) * _PIX, _PIX)
        off_b = pl.multiple_of((cam * _NCH + c0 + 1) * _PIX, _PIX)
        for off, tbl in ((off_a, tbl_a), (off_b, tbl_b)):
            cp = pltpu.make_async_copy(
                x_hbm.at[pl.ds(off, _PIX)],
                tbl.at[pl.ds(cam * _PIX, _PIX)], sem_tbl)
            cp.start()
            staging.append(cp)
    zeros = jnp.zeros((16,), jnp.float32)
    tbl_a[pl.ds(_NSRC, 16)] = zeros
    tbl_b[pl.ds(_NSRC, 16)] = zeros

    base_a = pl.multiple_of(c0 * _NCELL, 8)
    base_b = pl.multiple_of((c0 + 1) * _NCELL, 8)

    def in_copy(k, b):
        return pltpu.make_async_copy(
            idx_hbm.at[pl.ds(k * _CHUNK, _CHUNK)], idx_bufs[b], sem_ins[b])

    def out_copy(ref, b, base, k):
        return pltpu.make_async_copy(
            ref, out_hbm.at[pl.ds(base + k * _CHUNK, _CHUNK)], sem_outs[b])

    in_copy(0, 0).start()
    for cp in staging:
        cp.wait()

    for k in range(_NCHUNK):
        b = k & 1
        if k + 1 < _NCHUNK:
            in_copy(k + 1, 1 - b).start()
        in_copy(k, b).wait()
        if k >= 2:
            out_copy(outa_bufs[b], b, base_a, k - 2).wait()
            out_copy(outb_bufs[b], b, base_b, k - 2).wait()
        idx_v, out_a, out_b = idx_bufs[b], outa_bufs[b], outb_bufs[b]

        @plsc.parallel_loop(0, _CHUNK, step=16, unroll=_UNROLL)
        def step(o):
            iv = idx_v[pl.ds(o, 16)]
            out_a[pl.ds(o, 16)] = plsc.load_gather(tbl_a, [iv])
            out_b[pl.ds(o, 16)] = plsc.load_gather(tbl_b, [iv])
        out_copy(out_a, b, base_a, k).start()
        out_copy(out_b, b, base_b, k).start()

    for k in (_NCHUNK - 2, _NCHUNK - 1):
        b = k & 1
        out_copy(outa_bufs[b], b, base_a, k).wait()
        out_copy(outb_bufs[b], b, base_b, k).wait()


def kernel(x):
    x1 = jnp.reshape(jnp.asarray(x, jnp.float32), (_NCAM * _NCH * _PIX,))
    out = _bev_gather_kernel()(x1, jnp.asarray(_IDX_MAP))
    return jnp.reshape(out, (1, _NCH, _NX0, _NX1))


# speedup vs baseline: 127.5795x; 4.8812x over previous
import functools

import jax
import jax.numpy as jnp
import numpy as np
from jax import lax
from jax.experimental import pallas as pl
from jax.experimental.pallas import tpu as pltpu
from jax.experimental.pallas import tpu_sc as plsc

_NX0, _NX1 = 600, 300
_NCAM, _NCH = 6, 64
_FH, _FW = 32, 64
_PIX = _FH * _FW
_NSRC = _NCAM * _PIX
_D = 71

_NT0 = 5
_ROWW = _NT0 * 128
_CPT = 8
_NCT = _NCH // _CPT
_NXG = 32 // _NCT
_RPW = _NX1 // _NXG
_GRP = _ROWW // 16
_TBL = _NSRC + 16


def _winner_map() -> np.ndarray:
    ds = np.arange(4.0, 75.0, 1.0, dtype=np.float32).reshape(-1, 1, 1) * np.ones(
        (1, _FH, _FW), np.float32)
    xs = np.linspace(0, 1023, _FW, dtype=np.float32).reshape(1, 1, _FW) * np.ones(
        (_D, _FH, 1), np.float32)
    ys = np.linspace(0, 511, _FH, dtype=np.float32).reshape(1, _FH, 1) * np.ones(
        (_D, 1, _FW), np.float32)

    matrix = np.asarray([[2019.613635, 1745.881668, -111.4337968, -419.9388818],
                         [26.01936737, 870.7969811, -2038.300785, -120.9971104],
                         [0.02443084799, 0.997614078, -0.06457000164, -0.006415358346]])
    m = np.vstack([matrix, np.asarray([0.0, 0.0, 0.0, 1.0])])
    inv_m = np.linalg.inv(m).astype(np.float32)

    c1700 = np.float32(1700.0)
    c512 = np.float32(512.0)
    c3517 = np.float32(3517.0)
    c1024 = np.float32(1024.0)
    p4 = np.stack([
        (xs * ds) * c1700 / c512,
        (ys * ds) * c3517 / c1024,
        ds,
        np.ones_like(ds),
    ], -1).reshape(-1, 4)

    geom = np.asarray(jnp.dot(jnp.asarray(p4), jnp.asarray(inv_m.T)))[:, :3]

    dx = np.array([0.25, 0.25, 20.0], dtype=np.float32)
    bx = np.array([-75.0 + 0.125, 0.0 + 0.125, -10.0 + 10.0], dtype=np.float32)
    gf = ((geom - (bx - dx / np.float32(2.0))) / dx).astype(np.int32)

    kept = ((gf[:, 0] >= 0) & (gf[:, 0] < _NX0)
            & (gf[:, 1] >= 0) & (gf[:, 1] < _NX1)
            & (gf[:, 2] >= 0) & (gf[:, 2] < 1))
    rank = gf[:, 0].astype(np.int64) * _NX1 + gf[:, 1]

    q = np.arange(_D * _PIX, dtype=np.int64)
    winner_q = np.full(_NX0 * _NX1, -1, dtype=np.int64)
    np.maximum.at(winner_q, rank[kept], q[kept])

    d_win = winner_q // _PIX
    cam = (5 * _D + d_win) % _NCAM
    src = (cam * _PIX + winner_q % _PIX).astype(np.int32)
    idx_map = np.where(winner_q >= 0, src, np.int32(_NSRC)).astype(np.int32)
    return idx_map


def _x1_major_map(idx_map: np.ndarray) -> np.ndarray:
    wm = idx_map.reshape(_NX0, _NX1)
    out = np.full((_NX1, _ROWW), np.int32(_NSRC), np.int32)
    out[:, :_NX0] = wm.T
    return out.reshape(-1)


_IDX_MAP = _x1_major_map(_winner_map())


@functools.cache
def _bev_gather_kernel():
    mesh = plsc.VectorSubcoreMesh(
        core_axis_name="c", subcore_axis_name="s", num_cores=2, num_subcores=16)
    return functools.partial(
        pl.kernel,
        out_type=jax.ShapeDtypeStruct((1, _NX1, _NCH, _NX0), jnp.float32),
        mesh=mesh,
        scratch_types=[
            [pltpu.VMEM((_TBL,), jnp.float32) for _ in range(_CPT)],
            pltpu.VMEM((_ROWW,), jnp.int32),
            pltpu.VMEM((_ROWW,), jnp.int32),
            pltpu.VMEM((_NT0, _CPT, 128), jnp.float32),
            pltpu.VMEM((_NT0, _CPT, 128), jnp.float32),
            pltpu.SemaphoreType.DMA,
            pltpu.SemaphoreType.DMA,
            pltpu.SemaphoreType.DMA,
            pltpu.SemaphoreType.DMA,
            pltpu.SemaphoreType.DMA,
        ],
        compiler_params=pltpu.CompilerParams(
            needs_layout_passes=False, disable_bounds_checks=True),
    )(_bev_gather_body)


def _bev_gather_body(x_hbm, idx_hbm, out_hbm, tbls,
                     idx0, idx1, outt0, outt1,
                     sem_in0, sem_in1, sem_out0, sem_out1, sem_tbl):
    idx_bufs = (idx0, idx1)
    out_bufs = (outt0, outt1)
    sem_ins = (sem_in0, sem_in1)
    sem_outs = (sem_out0, sem_out1)
    wid = lax.axis_index("s") * 2 + lax.axis_index("c")
    ct = lax.bitwise_and(wid, _NCT - 1)
    xg = lax.shift_right_logical(wid, 3)
    c0 = ct * _CPT
    r0 = xg * _RPW

    staging = []
    for i in range(_CPT):
        for cam in range(_NCAM):
            off = pl.multiple_of((cam * _NCH + c0 + i) * _PIX, _PIX)
            cp = pltpu.make_async_copy(
                x_hbm.at[pl.ds(off, _PIX)],
                tbls[i].at[pl.ds(cam * _PIX, _PIX)], sem_tbl)
            cp.start()
            staging.append(cp)
    zeros = jnp.zeros((16,), jnp.float32)
    for i in range(_CPT):
        tbls[i][pl.ds(_NSRC, 16)] = zeros

    def in_copy(r, b):
        off = pl.multiple_of((r0 + r) * _ROWW, _ROWW)
        return pltpu.make_async_copy(
            idx_hbm.at[pl.ds(off, _ROWW)], idx_bufs[b], sem_ins[b])

    def out_copies(ref, b, r):
        cps = []
        for t in range(_NT0):
            lane0 = pl.multiple_of(wid * 0 + t * 128, 128)
            cps.append(pltpu.make_async_copy(
                ref.at[t],
                out_hbm.at[0, r0 + r, pl.ds(c0, _CPT), pl.ds(lane0, 128)],
                sem_outs[b]))
        return cps

    def gather_row(b):
        idx_v, out_t = idx_bufs[b], out_bufs[b]

        @plsc.parallel_loop(0, _GRP, step=1, unroll=4)
        def step(g):
            t = lax.shift_right_logical(g, 3)
            jg = lax.shift_left(lax.bitwise_and(g, 7), 4)
            iv = idx_v[pl.ds(g * 16, 16)]
            for i in range(_CPT):
                out_t[t, i, pl.ds(jg, 16)] = plsc.load_gather(tbls[i], [iv])

    in_copy(0, 0).start()
    in_copy(1, 1).start()
    for cp in staging:
        cp.wait()

    for r in (0, 1):
        b = r & 1
        in_copy(r, b).wait()
        gather_row(b)
        for cp in out_copies(out_bufs[b], b, r):
            cp.start()
        in_copy(r + 2, b).start()

    def double_row(k, carry):
        for b in (0, 1):
            r = 2 * k + b
            in_copy(r, b).wait()
            for cp in out_copies(out_bufs[b], b, r - 2):
                cp.wait()
            gather_row(b)
            for cp in out_copies(out_bufs[b], b, r):
                cp.start()
            in_copy(jnp.minimum(r + 2, _RPW - 1), b).start()
        return carry

    lax.fori_loop(1, _RPW // 2, double_row, 0)

    in_copy(_RPW - 1, 0).wait()
    for cp in out_copies(out_bufs[0], 0, _RPW - 3):
        cp.wait()
    gather_row(0)
    for cp in out_copies(out_bufs[0], 0, _RPW - 1):
        cp.start()

    in_copy(_RPW - 1, 1).wait()
    for cp in out_copies(out_bufs[1], 1, _RPW - 2):
        cp.wait()
    for cp in out_copies(out_bufs[0], 0, _RPW - 1):
        cp.wait()


def kernel(x):
    x1 = jnp.reshape(jnp.asarray(x, jnp.float32), (_NCAM * _NCH * _PIX,))
    out = _bev_gather_kernel()(x1, jnp.asarray(_IDX_MAP))
    return jnp.transpose(out, (0, 2, 3, 1))


# speedup vs baseline: 144.9265x; 1.1360x over previous
import functools

import jax
import jax.numpy as jnp
import numpy as np
from jax import lax
from jax.experimental import pallas as pl
from jax.experimental.pallas import tpu as pltpu
from jax.experimental.pallas import tpu_sc as plsc

_NX0, _NX1 = 600, 300
_NCAM, _NCH = 6, 64
_FH, _FW = 32, 64
_PIX = _FH * _FW
_NSRC = _NCAM * _PIX
_D = 71

_NT0 = 5
_ROWW = _NT0 * 128
_CPT = 8
_NCT = _NCH // _CPT
_NXG = 32 // _NCT
_RPW = _NX1 // _NXG
_GRP = _ROWW // 16
_NBUF = 4
_TBL = _NSRC + 16


def _winner_map() -> np.ndarray:
    ds = np.arange(4.0, 75.0, 1.0, dtype=np.float32).reshape(-1, 1, 1) * np.ones(
        (1, _FH, _FW), np.float32)
    xs = np.linspace(0, 1023, _FW, dtype=np.float32).reshape(1, 1, _FW) * np.ones(
        (_D, _FH, 1), np.float32)
    ys = np.linspace(0, 511, _FH, dtype=np.float32).reshape(1, _FH, 1) * np.ones(
        (_D, 1, _FW), np.float32)

    matrix = np.asarray([[2019.613635, 1745.881668, -111.4337968, -419.9388818],
                         [26.01936737, 870.7969811, -2038.300785, -120.9971104],
                         [0.02443084799, 0.997614078, -0.06457000164, -0.006415358346]])
    m = np.vstack([matrix, np.asarray([0.0, 0.0, 0.0, 1.0])])
    inv_m = np.linalg.inv(m).astype(np.float32)

    c1700 = np.float32(1700.0)
    c512 = np.float32(512.0)
    c3517 = np.float32(3517.0)
    c1024 = np.float32(1024.0)
    p4 = np.stack([
        (xs * ds) * c1700 / c512,
        (ys * ds) * c3517 / c1024,
        ds,
        np.ones_like(ds),
    ], -1).reshape(-1, 4)

    geom = np.asarray(jnp.dot(jnp.asarray(p4), jnp.asarray(inv_m.T)))[:, :3]

    dx = np.array([0.25, 0.25, 20.0], dtype=np.float32)
    bx = np.array([-75.0 + 0.125, 0.0 + 0.125, -10.0 + 10.0], dtype=np.float32)
    gf = ((geom - (bx - dx / np.float32(2.0))) / dx).astype(np.int32)

    kept = ((gf[:, 0] >= 0) & (gf[:, 0] < _NX0)
            & (gf[:, 1] >= 0) & (gf[:, 1] < _NX1)
            & (gf[:, 2] >= 0) & (gf[:, 2] < 1))
    rank = gf[:, 0].astype(np.int64) * _NX1 + gf[:, 1]

    q = np.arange(_D * _PIX, dtype=np.int64)
    winner_q = np.full(_NX0 * _NX1, -1, dtype=np.int64)
    np.maximum.at(winner_q, rank[kept], q[kept])

    d_win = winner_q // _PIX
    cam = (5 * _D + d_win) % _NCAM
    src = (cam * _PIX + winner_q % _PIX).astype(np.int32)
    idx_map = np.where(winner_q >= 0, src, np.int32(_NSRC)).astype(np.int32)
    return idx_map


def _x1_major_map(idx_map: np.ndarray) -> np.ndarray:
    wm = idx_map.reshape(_NX0, _NX1)
    out = np.full((_NX1, _ROWW), np.int32(_NSRC), np.int32)
    out[:, :_NX0] = wm.T
    return out.reshape(-1)


_IDX_MAP = _x1_major_map(_winner_map())


@functools.cache
def _bev_gather_kernel():
    mesh = plsc.VectorSubcoreMesh(
        core_axis_name="c", subcore_axis_name="s", num_cores=2, num_subcores=16)
    return functools.partial(
        pl.kernel,
        out_type=jax.ShapeDtypeStruct((1, _NX1, _NCH, _NX0), jnp.float32),
        mesh=mesh,
        scratch_types=[
            [pltpu.VMEM((_TBL,), jnp.float32) for _ in range(_CPT)],
            [pltpu.VMEM((_ROWW,), jnp.int32) for _ in range(_NBUF)],
            [pltpu.VMEM((_NT0, _CPT, 128), jnp.float32) for _ in range(_NBUF)],
            [pltpu.SemaphoreType.DMA for _ in range(_NBUF)],
            [pltpu.SemaphoreType.DMA for _ in range(_NBUF)],
            pltpu.SemaphoreType.DMA,
        ],
        compiler_params=pltpu.CompilerParams(
            needs_layout_passes=False, disable_bounds_checks=True),
    )(_bev_gather_body)


def _bev_gather_body(x_hbm, idx_hbm, out_hbm, tbls,
                     idx_bufs, out_bufs, sem_ins, sem_outs, sem_tbl):
    wid = lax.axis_index("s") * 2 + lax.axis_index("c")
    ct = lax.bitwise_and(wid, _NCT - 1)
    xg = lax.shift_right_logical(wid, 3)
    c0 = ct * _CPT
    r0 = xg * _RPW

    staging = []
    for i in range(_CPT):
        for cam in range(_NCAM):
            off = pl.multiple_of((cam * _NCH + c0 + i) * _PIX, _PIX)
            cp = pltpu.make_async_copy(
                x_hbm.at[pl.ds(off, _PIX)],
                tbls[i].at[pl.ds(cam * _PIX, _PIX)], sem_tbl)
            cp.start()
            staging.append(cp)
    zeros = jnp.zeros((16,), jnp.float32)
    for i in range(_CPT):
        tbls[i][pl.ds(_NSRC, 16)] = zeros

    def in_copy(r, b):
        off = pl.multiple_of((r0 + r) * _ROWW, _ROWW)
        return pltpu.make_async_copy(
            idx_hbm.at[pl.ds(off, _ROWW)], idx_bufs[b], sem_ins[b])

    def out_copies(ref, b, r):
        cps = []
        for t in range(_NT0):
            lane0 = pl.multiple_of(wid * 0 + t * 128, 128)
            cps.append(pltpu.make_async_copy(
                ref.at[t],
                out_hbm.at[0, r0 + r, pl.ds(c0, _CPT), pl.ds(lane0, 128)],
                sem_outs[b]))
        return cps

    def gather_row(b):
        idx_v, out_t = idx_bufs[b], out_bufs[b]

        @plsc.parallel_loop(0, _GRP, step=1, unroll=8)
        def step(g):
            t = lax.shift_right_logical(g, 3)
            jg = lax.shift_left(lax.bitwise_and(g, 7), 4)
            iv = idx_v[pl.ds(g * 16, 16)]
            for i in range(_CPT):
                out_t[t, i, pl.ds(jg, 16)] = plsc.load_gather(tbls[i], [iv])

    for b in range(_NBUF):
        in_copy(b, b).start()
    for cp in staging:
        cp.wait()

    for r in range(_NBUF):
        b = r
        in_copy(r, b).wait()
        gather_row(b)
        for cp in out_copies(out_bufs[b], b, r):
            cp.start()
        in_copy(r + _NBUF, b).start()

    def block(k, carry):
        for b in range(_NBUF):
            r = _NBUF * k + b
            in_copy(r, b).wait()
            for cp in out_copies(out_bufs[b], b, r - _NBUF):
                cp.wait()
            gather_row(b)
            for cp in out_copies(out_bufs[b], b, r):
                cp.start()
            in_copy(jnp.minimum(r + _NBUF, _RPW - 1), b).start()
        return carry

    nblk = _RPW // _NBUF
    lax.fori_loop(1, nblk, block, 0)

    for r in range(_NBUF * nblk, _RPW):
        b = r - _NBUF * nblk
        in_copy(r, b).wait()
        for cp in out_copies(out_bufs[b], b, r - _NBUF):
            cp.wait()
        gather_row(b)
        for cp in out_copies(out_bufs[b], b, r):
            cp.start()

    for b in range(_RPW - _NBUF * nblk, _NBUF):
        in_copy(_RPW - 1, b).wait()
    for r in range(_RPW - _NBUF, _RPW):
        b = r % _NBUF
        for cp in out_copies(out_bufs[b], b, r):
            cp.wait()


def kernel(x):
    x1 = jnp.reshape(jnp.asarray(x, jnp.float32), (_NCAM * _NCH * _PIX,))
    out = _bev_gather_kernel()(x1, jnp.asarray(_IDX_MAP))
    return jnp.transpose(out, (0, 2, 3, 1))
